# baseline (speedup 1.0000x reference)
import functools

import jax
import jax.numpy as jnp
from jax import lax
from jax.experimental import pallas as pl
from jax.experimental.pallas import tpu as pltpu
from jax.experimental.pallas import tpu_sc as plsc

N = 10000
E = 160000
IN_C = 64
OUT_C = 32
EDGE_DIM = 16
HIDDEN = 64
AUG = 40

NC = 2
NS = 16
NW = NC * NS
CH = 128
E_PAD = 163840
EPW = E_PAD // NW
NCHUNK = EPW // CH
N_ACC = 10016
RPZ = N_ACC // NS


def _sc_gather(x, src_pad):
    mesh = plsc.VectorSubcoreMesh(core_axis_name="c", subcore_axis_name="s")

    @functools.partial(
        pl.kernel,
        mesh=mesh,
        out_type=jax.ShapeDtypeStruct((E_PAD, IN_C), jnp.float32),
        compiler_params=pltpu.CompilerParams(use_tc_tiling_on_sc=False),
        scratch_types=[
            pltpu.VMEM((CH,), jnp.int32),
            pltpu.VMEM((CH, IN_C), jnp.float32),
            pltpu.SemaphoreType.DMA,
        ],
    )
    def k(x_hbm, src_hbm, out_hbm, idx_v, rows_v, sem):
        c = lax.axis_index("c")
        s = lax.axis_index("s")
        base = (s * NC + c) * EPW

        def body(j, carry):
            off = base + j * CH
            pltpu.sync_copy(src_hbm.at[pl.ds(off, CH)], idx_v)
            pltpu.async_copy(x_hbm.at[idx_v], rows_v, sem).wait()
            pltpu.sync_copy(rows_v, out_hbm.at[pl.ds(off, CH)])
            return carry

        lax.fori_loop(0, NCHUNK, body, 0)

    return k(x, src_pad)


def _sc_scatter(msgs_aug, dst2, zrows):
    mesh = plsc.VectorSubcoreMesh(core_axis_name="c", subcore_axis_name="s")

    @functools.partial(
        pl.kernel,
        mesh=mesh,
        out_type=jax.ShapeDtypeStruct((2, N_ACC, AUG), jnp.float32),
        compiler_params=pltpu.CompilerParams(use_tc_tiling_on_sc=False),
        scratch_types=[
            pltpu.VMEM((NCHUNK, CH), jnp.int32),
            pltpu.VMEM((CH, AUG), jnp.float32),
            pltpu.VMEM_SHARED((N_ACC, AUG), jnp.float32),
        ],
    )
    def k(m_hbm, dst_hbm, z_hbm, out_hbm, idx_v, rows_v, acc_sh):
        c = lax.axis_index("c")
        s = lax.axis_index("s")
        pltpu.sync_copy(z_hbm.at[pl.ds(s * RPZ, RPZ)],
                        acc_sh.at[pl.ds(s * RPZ, RPZ)])
        plsc.subcore_barrier()

        wid = s * NC + c
        pltpu.sync_copy(dst_hbm.at[pl.ds(wid * NCHUNK, NCHUNK)], idx_v)

        def body(j, carry):
            off = wid * EPW + j * CH
            pltpu.sync_copy(m_hbm.at[pl.ds(off, CH)], rows_v)
            pltpu.sync_copy(rows_v, acc_sh.at[idx_v.at[j]], add=True)
            return carry

        lax.fori_loop(0, NCHUNK, body, 0)
        plsc.subcore_barrier()
        pltpu.sync_copy(acc_sh.at[pl.ds(s * RPZ, RPZ)],
                        out_hbm.at[c, pl.ds(s * RPZ, RPZ)])

    return k(msgs_aug, dst2, zrows)


BE = 512


def _edge_body(ea_ref, xj_ref, W1_ref, b1_ref, W2_ref, b2_ref, o_ref):
    h = jnp.dot(ea_ref[...], W1_ref[...], preferred_element_type=jnp.float32)
    h = h + b1_ref[...]
    h = h * jax.nn.sigmoid(h)
    q = jnp.dot(h, W2_ref[...], preferred_element_type=jnp.float32)
    q = q + b2_ref[...]
    xj = xj_ref[...]
    acc = jnp.zeros((BE, OUT_C), jnp.float32)
    for i in range(IN_C):
        acc = acc + xj[:, i:i + 1] * q[:, i * OUT_C:(i + 1) * OUT_C]
    o_ref[:, 0:OUT_C] = acc
    o_ref[:, OUT_C:OUT_C + 1] = jnp.ones((BE, 1), jnp.float32)
    o_ref[:, OUT_C + 1:AUG] = jnp.zeros((BE, AUG - OUT_C - 1), jnp.float32)


def _tc_edge(ea_pad, xj, W1, b1, W2, b2):
    grid = (E_PAD // BE,)
    return pl.pallas_call(
        _edge_body,
        grid=grid,
        in_specs=[
            pl.BlockSpec((BE, EDGE_DIM), lambda i: (i, 0)),
            pl.BlockSpec((BE, IN_C), lambda i: (i, 0)),
            pl.BlockSpec((EDGE_DIM, HIDDEN), lambda i: (0, 0)),
            pl.BlockSpec((1, HIDDEN), lambda i: (0, 0)),
            pl.BlockSpec((HIDDEN, IN_C * OUT_C), lambda i: (0, 0)),
            pl.BlockSpec((1, IN_C * OUT_C), lambda i: (0, 0)),
        ],
        out_specs=pl.BlockSpec((BE, AUG), lambda i: (i, 0)),
        out_shape=jax.ShapeDtypeStruct((E_PAD, AUG), jnp.float32),
    )(ea_pad, xj, W1, b1.reshape(1, HIDDEN), W2, b2.reshape(1, IN_C * OUT_C))


BN = 2000


def _node_body(p_ref, x_ref, root_ref, bias_ref, g_ref, beta_ref,
               Wres_ref, bres_ref, o_ref):
    p = p_ref[0] + p_ref[1]
    summed = p[:, 0:OUT_C]
    cnt = p[:, OUT_C:OUT_C + 1]
    aggr = summed / jnp.maximum(cnt, 1.0)
    xb = x_ref[...]
    out = aggr + jnp.dot(xb, root_ref[...],
                         preferred_element_type=jnp.float32) + bias_ref[...]
    mu = jnp.mean(out, axis=1, keepdims=True)
    var = jnp.mean((out - mu) * (out - mu), axis=1, keepdims=True)
    out = (out - mu) * lax.rsqrt(var + 1e-5) * g_ref[...] + beta_ref[...]
    res = jnp.dot(xb, Wres_ref[...],
                  preferred_element_type=jnp.float32) + bres_ref[...]
    t = out + res
    o_ref[...] = t * jax.nn.sigmoid(t)


def _tc_node(partials, x, root, bias, ln_gamma, ln_beta, Wres, bres):
    grid = (N // BN,)
    return pl.pallas_call(
        _node_body,
        grid=grid,
        in_specs=[
            pl.BlockSpec((2, BN, AUG), lambda i: (0, i, 0)),
            pl.BlockSpec((BN, IN_C), lambda i: (i, 0)),
            pl.BlockSpec((IN_C, OUT_C), lambda i: (0, 0)),
            pl.BlockSpec((1, OUT_C), lambda i: (0, 0)),
            pl.BlockSpec((1, OUT_C), lambda i: (0, 0)),
            pl.BlockSpec((1, OUT_C), lambda i: (0, 0)),
            pl.BlockSpec((IN_C, OUT_C), lambda i: (0, 0)),
            pl.BlockSpec((1, OUT_C), lambda i: (0, 0)),
        ],
        out_specs=pl.BlockSpec((BN, OUT_C), lambda i: (i, 0)),
        out_shape=jax.ShapeDtypeStruct((N, OUT_C), jnp.float32),
    )(partials, x, root, bias.reshape(1, OUT_C), ln_gamma.reshape(1, OUT_C),
      ln_beta.reshape(1, OUT_C), Wres, bres.reshape(1, OUT_C))


def kernel(x, edge_attr, W1, b1, W2, b2, root, bias, ln_gamma, ln_beta,
           Wres, bres, edge_index):
    pad = E_PAD - E
    src_pad = jnp.concatenate(
        [edge_index[0], jnp.zeros((pad,), jnp.int32)])
    dst_pad = jnp.concatenate(
        [edge_index[1], jnp.full((pad,), N, jnp.int32)])
    ea_pad = jnp.concatenate(
        [edge_attr, jnp.zeros((pad, EDGE_DIM), jnp.float32)])
    dst2 = dst_pad.reshape(E_PAD // CH, CH)
    zrows = jnp.zeros((N_ACC, AUG), jnp.float32)

    xj = _sc_gather(x, src_pad)
    msgs_aug = _tc_edge(ea_pad, xj, W1, b1, W2, b2)
    partials = _sc_scatter(msgs_aug, dst2, zrows)
    return _tc_node(partials, x, root, bias, ln_gamma, ln_beta, Wres, bres)


# speedup vs baseline: 3.1909x; 3.1909x over previous
import functools

import jax
import jax.numpy as jnp
from jax import lax
from jax.experimental import pallas as pl
from jax.experimental.pallas import tpu as pltpu
from jax.experimental.pallas import tpu_sc as plsc

N = 10000
E = 160000
IN_C = 64
OUT_C = 32
EDGE_DIM = 16
HIDDEN = 64
AUG = 40

NC = 2
NS = 16
NW = NC * NS
CH = 128
E_PAD = 163840
EPW = E_PAD // NW
NCHUNK = EPW // CH
N_ACC = 10016
RPZ = N_ACC // NS


def _sc_gather(x, src_pad):
    mesh = plsc.VectorSubcoreMesh(core_axis_name="c", subcore_axis_name="s")

    @functools.partial(
        pl.kernel,
        mesh=mesh,
        out_type=jax.ShapeDtypeStruct((E_PAD, IN_C), jnp.float32),
        compiler_params=pltpu.CompilerParams(use_tc_tiling_on_sc=False),
        scratch_types=[
            pltpu.VMEM((CH,), jnp.int32),
            pltpu.VMEM((CH, IN_C), jnp.float32),
            pltpu.SemaphoreType.DMA,
        ],
    )
    def k(x_hbm, src_hbm, out_hbm, idx_v, rows_v, sem):
        c = lax.axis_index("c")
        s = lax.axis_index("s")
        base = (s * NC + c) * EPW

        def body(j, carry):
            off = base + j * CH
            pltpu.sync_copy(src_hbm.at[pl.ds(off, CH)], idx_v)
            pltpu.async_copy(x_hbm.at[idx_v], rows_v, sem).wait()
            pltpu.sync_copy(rows_v, out_hbm.at[pl.ds(off, CH)])
            return carry

        lax.fori_loop(0, NCHUNK, body, 0)

    return k(x, src_pad)


def _sc_scatter(msgs_aug, dst2, zrows):
    mesh = plsc.VectorSubcoreMesh(core_axis_name="c", subcore_axis_name="s")

    @functools.partial(
        pl.kernel,
        mesh=mesh,
        out_type=jax.ShapeDtypeStruct((2, N_ACC, AUG), jnp.float32),
        compiler_params=pltpu.CompilerParams(use_tc_tiling_on_sc=False),
        scratch_types=[
            pltpu.VMEM((NCHUNK, CH), jnp.int32),
            pltpu.VMEM((CH, AUG), jnp.float32),
            pltpu.VMEM_SHARED((N_ACC, AUG), jnp.float32),
        ],
    )
    def k(m_hbm, dst_hbm, z_hbm, out_hbm, idx_v, rows_v, acc_sh):
        c = lax.axis_index("c")
        s = lax.axis_index("s")
        pltpu.sync_copy(z_hbm.at[pl.ds(s * RPZ, RPZ)],
                        acc_sh.at[pl.ds(s * RPZ, RPZ)])
        plsc.subcore_barrier()

        wid = s * NC + c
        pltpu.sync_copy(dst_hbm.at[pl.ds(wid * NCHUNK, NCHUNK)], idx_v)

        def body(j, carry):
            off = wid * EPW + j * CH
            pltpu.sync_copy(m_hbm.at[pl.ds(off, CH)], rows_v)
            pltpu.sync_copy(rows_v, acc_sh.at[idx_v.at[j]], add=True)
            return carry

        lax.fori_loop(0, NCHUNK, body, 0)
        plsc.subcore_barrier()
        pltpu.sync_copy(acc_sh.at[pl.ds(s * RPZ, RPZ)],
                        out_hbm.at[c, pl.ds(s * RPZ, RPZ)])

    return k(msgs_aug, dst2, zrows)


BE = 512


def _edge_body(ea_ref, xj_ref, W1_ref, b1_ref, W2_ref, b2_ref, R_ref, S_ref,
               o_ref):
    h = jnp.dot(ea_ref[...], W1_ref[...], preferred_element_type=jnp.float32)
    h = h + b1_ref[...]
    h = h * jax.nn.sigmoid(h)
    q = jnp.dot(h, W2_ref[...], preferred_element_type=jnp.float32)
    q = q + b2_ref[...]
    xje = jnp.dot(xj_ref[...], R_ref[...], preferred_element_type=jnp.float32)
    p = q * xje
    p = p[:, :1024] + p[:, 1024:]
    p = p[:, :512] + p[:, 512:]
    p = p[:, :256] + p[:, 256:]
    p = p[:, :128] + p[:, 128:]
    acc = jnp.dot(p, S_ref[...], preferred_element_type=jnp.float32)
    o_ref[:, 0:OUT_C] = acc
    o_ref[:, OUT_C:OUT_C + 1] = jnp.ones((BE, 1), jnp.float32)
    o_ref[:, OUT_C + 1:AUG] = jnp.zeros((BE, AUG - OUT_C - 1), jnp.float32)


def _tc_edge(ea_pad, xj, W1, b1, W2, b2):
    grid = (E_PAD // BE,)
    R = jnp.kron(jnp.eye(IN_C, dtype=jnp.float32),
                 jnp.ones((1, OUT_C), jnp.float32))
    S = jnp.tile(jnp.eye(OUT_C, dtype=jnp.float32), (4, 1))
    return pl.pallas_call(
        _edge_body,
        grid=grid,
        in_specs=[
            pl.BlockSpec((BE, EDGE_DIM), lambda i: (i, 0)),
            pl.BlockSpec((BE, IN_C), lambda i: (i, 0)),
            pl.BlockSpec((EDGE_DIM, HIDDEN), lambda i: (0, 0)),
            pl.BlockSpec((1, HIDDEN), lambda i: (0, 0)),
            pl.BlockSpec((HIDDEN, IN_C * OUT_C), lambda i: (0, 0)),
            pl.BlockSpec((1, IN_C * OUT_C), lambda i: (0, 0)),
            pl.BlockSpec((IN_C, IN_C * OUT_C), lambda i: (0, 0)),
            pl.BlockSpec((4 * OUT_C, OUT_C), lambda i: (0, 0)),
        ],
        out_specs=pl.BlockSpec((BE, AUG), lambda i: (i, 0)),
        out_shape=jax.ShapeDtypeStruct((E_PAD, AUG), jnp.float32),
    )(ea_pad, xj, W1, b1.reshape(1, HIDDEN), W2, b2.reshape(1, IN_C * OUT_C),
      R, S)


BN = 2000


def _node_body(p_ref, x_ref, root_ref, bias_ref, g_ref, beta_ref,
               Wres_ref, bres_ref, o_ref):
    p = p_ref[0] + p_ref[1]
    summed = p[:, 0:OUT_C]
    cnt = p[:, OUT_C:OUT_C + 1]
    aggr = summed / jnp.maximum(cnt, 1.0)
    xb = x_ref[...]
    out = aggr + jnp.dot(xb, root_ref[...],
                         preferred_element_type=jnp.float32) + bias_ref[...]
    mu = jnp.mean(out, axis=1, keepdims=True)
    var = jnp.mean((out - mu) * (out - mu), axis=1, keepdims=True)
    out = (out - mu) * lax.rsqrt(var + 1e-5) * g_ref[...] + beta_ref[...]
    res = jnp.dot(xb, Wres_ref[...],
                  preferred_element_type=jnp.float32) + bres_ref[...]
    t = out + res
    o_ref[...] = t * jax.nn.sigmoid(t)


def _tc_node(partials, x, root, bias, ln_gamma, ln_beta, Wres, bres):
    grid = (N // BN,)
    return pl.pallas_call(
        _node_body,
        grid=grid,
        in_specs=[
            pl.BlockSpec((2, BN, AUG), lambda i: (0, i, 0)),
            pl.BlockSpec((BN, IN_C), lambda i: (i, 0)),
            pl.BlockSpec((IN_C, OUT_C), lambda i: (0, 0)),
            pl.BlockSpec((1, OUT_C), lambda i: (0, 0)),
            pl.BlockSpec((1, OUT_C), lambda i: (0, 0)),
            pl.BlockSpec((1, OUT_C), lambda i: (0, 0)),
            pl.BlockSpec((IN_C, OUT_C), lambda i: (0, 0)),
            pl.BlockSpec((1, OUT_C), lambda i: (0, 0)),
        ],
        out_specs=pl.BlockSpec((BN, OUT_C), lambda i: (i, 0)),
        out_shape=jax.ShapeDtypeStruct((N, OUT_C), jnp.float32),
    )(partials, x, root, bias.reshape(1, OUT_C), ln_gamma.reshape(1, OUT_C),
      ln_beta.reshape(1, OUT_C), Wres, bres.reshape(1, OUT_C))


def kernel(x, edge_attr, W1, b1, W2, b2, root, bias, ln_gamma, ln_beta,
           Wres, bres, edge_index):
    pad = E_PAD - E
    src_pad = jnp.concatenate(
        [edge_index[0], jnp.zeros((pad,), jnp.int32)])
    dst_pad = jnp.concatenate(
        [edge_index[1], jnp.full((pad,), N, jnp.int32)])
    ea_pad = jnp.concatenate(
        [edge_attr, jnp.zeros((pad, EDGE_DIM), jnp.float32)])
    dst2 = dst_pad.reshape(E_PAD // CH, CH)
    zrows = jnp.zeros((N_ACC, AUG), jnp.float32)

    xj = _sc_gather(x, src_pad)
    msgs_aug = _tc_edge(ea_pad, xj, W1, b1, W2, b2)
    partials = _sc_scatter(msgs_aug, dst2, zrows)
    return _tc_node(partials, x, root, bias, ln_gamma, ln_beta, Wres, bres)


# speedup vs baseline: 3.6349x; 1.1391x over previous
import functools

import jax
import jax.numpy as jnp
from jax import lax
from jax.experimental import pallas as pl
from jax.experimental.pallas import tpu as pltpu
from jax.experimental.pallas import tpu_sc as plsc

N = 10000
E = 160000
IN_C = 64
OUT_C = 32
EDGE_DIM = 16
HIDDEN = 64
AUG = 40

NC = 2
NS = 16
NW = NC * NS
CH = 128
E_PAD = 163840
EPW = E_PAD // NW
NCHUNK = EPW // CH
N_ACC = 10016
RPZ = N_ACC // NS


NPIPE = 4
ECH = E_PAD // NPIPE
CPW = ECH // NW // CH


def _sc_gather_chunk(x, src2, ci):
    mesh = plsc.VectorSubcoreMesh(core_axis_name="c", subcore_axis_name="s")

    @functools.partial(
        pl.kernel,
        mesh=mesh,
        out_type=jax.ShapeDtypeStruct((ECH, IN_C), jnp.float32),
        compiler_params=pltpu.CompilerParams(use_tc_tiling_on_sc=False),
        scratch_types=[
            pltpu.VMEM((CPW, CH), jnp.int32),
            pltpu.VMEM((CH, IN_C), jnp.float32),
            pltpu.SemaphoreType.DMA,
        ],
    )
    def k(x_hbm, src2_hbm, out_hbm, idx2, rows_v, sem):
        c = lax.axis_index("c")
        s = lax.axis_index("s")
        wid = s * NC + c
        pltpu.sync_copy(
            src2_hbm.at[pl.ds(ci * (ECH // CH) + wid * CPW, CPW)], idx2)

        def body(j, carry):
            pltpu.async_copy(x_hbm.at[idx2.at[j]], rows_v, sem).wait()
            pltpu.sync_copy(
                rows_v, out_hbm.at[pl.ds(wid * CPW * CH + j * CH, CH)])
            return carry

        lax.fori_loop(0, CPW, body, 0)

    return k(x, src2)


def _sc_scatter(msgs_list, dst2, zrows):
    mesh = plsc.VectorSubcoreMesh(core_axis_name="c", subcore_axis_name="s")

    @functools.partial(
        pl.kernel,
        mesh=mesh,
        out_type=jax.ShapeDtypeStruct((2, N_ACC, AUG), jnp.float32),
        compiler_params=pltpu.CompilerParams(use_tc_tiling_on_sc=False),
        scratch_types=[
            pltpu.VMEM((CPW, CH), jnp.int32),
            pltpu.VMEM((CH, AUG), jnp.float32),
            pltpu.VMEM_SHARED((N_ACC, AUG), jnp.float32),
        ],
    )
    def k(m0, m1, m2, m3, dst_hbm, z_hbm, out_hbm, idx_v, rows_v, acc_sh):
        c = lax.axis_index("c")
        s = lax.axis_index("s")
        pltpu.sync_copy(z_hbm.at[pl.ds(s * RPZ, RPZ)],
                        acc_sh.at[pl.ds(s * RPZ, RPZ)])
        plsc.subcore_barrier()

        wid = s * NC + c
        for ci, m_hbm in enumerate((m0, m1, m2, m3)):
            pltpu.sync_copy(
                dst_hbm.at[pl.ds(ci * (ECH // CH) + wid * CPW, CPW)], idx_v)

            def body(j, carry, m_hbm=m_hbm):
                pltpu.sync_copy(
                    m_hbm.at[pl.ds(wid * CPW * CH + j * CH, CH)], rows_v)
                pltpu.sync_copy(rows_v, acc_sh.at[idx_v.at[j]], add=True)
                return carry

            lax.fori_loop(0, CPW, body, 0)
        plsc.subcore_barrier()
        pltpu.sync_copy(acc_sh.at[pl.ds(s * RPZ, RPZ)],
                        out_hbm.at[c, pl.ds(s * RPZ, RPZ)])

    return k(*msgs_list, dst2, zrows)


BE = 512


def _edge_body(ea_ref, xj_ref, W1_ref, b1_ref, W2_ref, b2_ref, R_ref, S_ref,
               o_ref):
    h = jnp.dot(ea_ref[...], W1_ref[...], preferred_element_type=jnp.float32)
    h = h + b1_ref[...]
    h = h * jax.nn.sigmoid(h)
    q = jnp.dot(h, W2_ref[...], preferred_element_type=jnp.float32)
    q = q + b2_ref[...]
    xje = jnp.dot(xj_ref[...], R_ref[...], preferred_element_type=jnp.float32)
    p = q * xje
    p = p[:, :1024] + p[:, 1024:]
    p = p[:, :512] + p[:, 512:]
    p = p[:, :256] + p[:, 256:]
    p = p[:, :128] + p[:, 128:]
    acc = jnp.dot(p, S_ref[...], preferred_element_type=jnp.float32)
    o_ref[:, 0:OUT_C] = acc
    o_ref[:, OUT_C:OUT_C + 1] = jnp.ones((BE, 1), jnp.float32)
    o_ref[:, OUT_C + 1:AUG] = jnp.zeros((BE, AUG - OUT_C - 1), jnp.float32)


def _tc_edge(ea_pad, xj, W1, b1, W2, b2, ci):
    grid = (ECH // BE,)
    off = ci * (ECH // BE)
    R = jnp.kron(jnp.eye(IN_C, dtype=jnp.float32),
                 jnp.ones((1, OUT_C), jnp.float32))
    S = jnp.tile(jnp.eye(OUT_C, dtype=jnp.float32), (4, 1))
    return pl.pallas_call(
        _edge_body,
        grid=grid,
        in_specs=[
            pl.BlockSpec((BE, EDGE_DIM), lambda i, off=off: (i + off, 0)),
            pl.BlockSpec((BE, IN_C), lambda i: (i, 0)),
            pl.BlockSpec((EDGE_DIM, HIDDEN), lambda i: (0, 0)),
            pl.BlockSpec((1, HIDDEN), lambda i: (0, 0)),
            pl.BlockSpec((HIDDEN, IN_C * OUT_C), lambda i: (0, 0)),
            pl.BlockSpec((1, IN_C * OUT_C), lambda i: (0, 0)),
            pl.BlockSpec((IN_C, IN_C * OUT_C), lambda i: (0, 0)),
            pl.BlockSpec((4 * OUT_C, OUT_C), lambda i: (0, 0)),
        ],
        out_specs=pl.BlockSpec((BE, AUG), lambda i: (i, 0)),
        out_shape=jax.ShapeDtypeStruct((ECH, AUG), jnp.float32),
    )(ea_pad, xj, W1, b1.reshape(1, HIDDEN), W2, b2.reshape(1, IN_C * OUT_C),
      R, S)


BN = 2000


def _node_body(p_ref, x_ref, root_ref, bias_ref, g_ref, beta_ref,
               Wres_ref, bres_ref, o_ref):
    p = p_ref[0] + p_ref[1]
    summed = p[:, 0:OUT_C]
    cnt = p[:, OUT_C:OUT_C + 1]
    aggr = summed / jnp.maximum(cnt, 1.0)
    xb = x_ref[...]
    out = aggr + jnp.dot(xb, root_ref[...],
                         preferred_element_type=jnp.float32) + bias_ref[...]
    mu = jnp.mean(out, axis=1, keepdims=True)
    var = jnp.mean((out - mu) * (out - mu), axis=1, keepdims=True)
    out = (out - mu) * lax.rsqrt(var + 1e-5) * g_ref[...] + beta_ref[...]
    res = jnp.dot(xb, Wres_ref[...],
                  preferred_element_type=jnp.float32) + bres_ref[...]
    t = out + res
    o_ref[...] = t * jax.nn.sigmoid(t)


def _tc_node(partials, x, root, bias, ln_gamma, ln_beta, Wres, bres):
    grid = (N // BN,)
    return pl.pallas_call(
        _node_body,
        grid=grid,
        in_specs=[
            pl.BlockSpec((2, BN, AUG), lambda i: (0, i, 0)),
            pl.BlockSpec((BN, IN_C), lambda i: (i, 0)),
            pl.BlockSpec((IN_C, OUT_C), lambda i: (0, 0)),
            pl.BlockSpec((1, OUT_C), lambda i: (0, 0)),
            pl.BlockSpec((1, OUT_C), lambda i: (0, 0)),
            pl.BlockSpec((1, OUT_C), lambda i: (0, 0)),
            pl.BlockSpec((IN_C, OUT_C), lambda i: (0, 0)),
            pl.BlockSpec((1, OUT_C), lambda i: (0, 0)),
        ],
        out_specs=pl.BlockSpec((BN, OUT_C), lambda i: (i, 0)),
        out_shape=jax.ShapeDtypeStruct((N, OUT_C), jnp.float32),
    )(partials, x, root, bias.reshape(1, OUT_C), ln_gamma.reshape(1, OUT_C),
      ln_beta.reshape(1, OUT_C), Wres, bres.reshape(1, OUT_C))


def kernel(x, edge_attr, W1, b1, W2, b2, root, bias, ln_gamma, ln_beta,
           Wres, bres, edge_index):
    pad = E_PAD - E
    src_pad = jnp.concatenate(
        [edge_index[0], jnp.zeros((pad,), jnp.int32)])
    dst_pad = jnp.concatenate(
        [edge_index[1], jnp.full((pad,), N, jnp.int32)])
    ea_pad = jnp.concatenate(
        [edge_attr, jnp.zeros((pad, EDGE_DIM), jnp.float32)])
    dst2 = dst_pad.reshape(E_PAD // CH, CH)
    src2 = src_pad.reshape(E_PAD // CH, CH)
    zrows = jnp.zeros((N_ACC, AUG), jnp.float32)

    msgs = []
    for ci in range(NPIPE):
        xj_c = _sc_gather_chunk(x, src2, ci)
        msgs.append(_tc_edge(ea_pad, xj_c, W1, b1, W2, b2, ci))
    partials = _sc_scatter(msgs, dst2, zrows)
    return _tc_node(partials, x, root, bias, ln_gamma, ln_beta, Wres, bres)


# speedup vs baseline: 3.7514x; 1.0321x over previous
import functools

import jax
import jax.numpy as jnp
from jax import lax
from jax.experimental import pallas as pl
from jax.experimental.pallas import tpu as pltpu
from jax.experimental.pallas import tpu_sc as plsc

N = 10000
E = 160000
IN_C = 64
OUT_C = 32
EDGE_DIM = 16
HIDDEN = 64
AUG = 40

NC = 2
NS = 16
NW = NC * NS
CH = 128
E_PAD = 163840
EPW = E_PAD // NW
NCHUNK = EPW // CH
N_ACC = 10016
RPZ = N_ACC // NS


NPIPE = 4
ECH = E_PAD // NPIPE
CPW = ECH // NW // CH


def _sc_gather_chunk(x, src2, ci):
    mesh = plsc.VectorSubcoreMesh(core_axis_name="c", subcore_axis_name="s")

    @functools.partial(
        pl.kernel,
        mesh=mesh,
        out_type=jax.ShapeDtypeStruct((ECH, IN_C), jnp.float32),
        compiler_params=pltpu.CompilerParams(use_tc_tiling_on_sc=False),
        scratch_types=[
            pltpu.VMEM((CPW, CH), jnp.int32),
            pltpu.VMEM((CPW * CH, IN_C), jnp.float32),
            pltpu.SemaphoreType.DMA,
        ],
    )
    def k(x_hbm, src2_hbm, out_hbm, idx2, rows_v, sem):
        c = lax.axis_index("c")
        s = lax.axis_index("s")
        wid = s * NC + c
        pltpu.sync_copy(
            src2_hbm.at[pl.ds(ci * (ECH // CH) + wid * CPW, CPW)], idx2)
        for j in range(CPW):
            pltpu.async_copy(x_hbm.at[idx2.at[j]],
                             rows_v.at[pl.ds(j * CH, CH)], sem)
        out_slice = out_hbm.at[pl.ds(wid * CPW * CH, CPW * CH)]
        pltpu.make_async_copy(out_slice, rows_v, sem).wait()
        pltpu.sync_copy(rows_v, out_slice)

    return k(x, src2)


def _sc_scatter(msgs_pair, dst2, zrows, cis):
    mesh = plsc.VectorSubcoreMesh(core_axis_name="c", subcore_axis_name="s")

    @functools.partial(
        pl.kernel,
        mesh=mesh,
        out_type=jax.ShapeDtypeStruct((2, N_ACC, AUG), jnp.float32),
        compiler_params=pltpu.CompilerParams(use_tc_tiling_on_sc=False),
        scratch_types=[
            pltpu.VMEM((CPW, CH), jnp.int32),
            pltpu.VMEM((CPW * CH, AUG), jnp.float32),
            pltpu.VMEM_SHARED((N_ACC, AUG), jnp.float32),
        ],
    )
    def k(m0, m1, dst_hbm, z_hbm, out_hbm, idx_v, rows_v, acc_sh):
        c = lax.axis_index("c")
        s = lax.axis_index("s")
        pltpu.sync_copy(z_hbm.at[pl.ds(s * RPZ, RPZ)],
                        acc_sh.at[pl.ds(s * RPZ, RPZ)])
        plsc.subcore_barrier()

        wid = s * NC + c
        for ci, m_hbm in zip(cis, (m0, m1)):
            pltpu.sync_copy(
                dst_hbm.at[pl.ds(ci * (ECH // CH) + wid * CPW, CPW)], idx_v)
            pltpu.sync_copy(
                m_hbm.at[pl.ds(wid * CPW * CH, CPW * CH)], rows_v)
            for j in range(CPW):
                pltpu.sync_copy(rows_v.at[pl.ds(j * CH, CH)],
                                acc_sh.at[idx_v.at[j]], add=True)
        plsc.subcore_barrier()
        pltpu.sync_copy(acc_sh.at[pl.ds(s * RPZ, RPZ)],
                        out_hbm.at[c, pl.ds(s * RPZ, RPZ)])

    return k(*msgs_pair, dst2, zrows)


BE = 512


def _edge_body(ea_ref, xj_ref, W1_ref, b1_ref, W2_ref, b2_ref, R_ref, S_ref,
               o_ref):
    h = jnp.dot(ea_ref[...], W1_ref[...], preferred_element_type=jnp.float32)
    h = h + b1_ref[...]
    h = h * jax.nn.sigmoid(h)
    q = jnp.dot(h, W2_ref[...], preferred_element_type=jnp.float32)
    q = q + b2_ref[...]
    xje = jnp.dot(xj_ref[...], R_ref[...], preferred_element_type=jnp.float32)
    p = q * xje
    p = p[:, :1024] + p[:, 1024:]
    p = p[:, :512] + p[:, 512:]
    p = p[:, :256] + p[:, 256:]
    p = p[:, :128] + p[:, 128:]
    acc = jnp.dot(p, S_ref[...], preferred_element_type=jnp.float32)
    o_ref[:, 0:OUT_C] = acc
    o_ref[:, OUT_C:OUT_C + 1] = jnp.ones((BE, 1), jnp.float32)
    o_ref[:, OUT_C + 1:AUG] = jnp.zeros((BE, AUG - OUT_C - 1), jnp.float32)


def _tc_edge(ea_pad, xj, W1, b1, W2, b2, ci):
    grid = (ECH // BE,)
    off = ci * (ECH // BE)
    R = jnp.kron(jnp.eye(IN_C, dtype=jnp.float32),
                 jnp.ones((1, OUT_C), jnp.float32))
    S = jnp.tile(jnp.eye(OUT_C, dtype=jnp.float32), (4, 1))
    return pl.pallas_call(
        _edge_body,
        grid=grid,
        in_specs=[
            pl.BlockSpec((BE, EDGE_DIM), lambda i, off=off: (i + off, 0)),
            pl.BlockSpec((BE, IN_C), lambda i: (i, 0)),
            pl.BlockSpec((EDGE_DIM, HIDDEN), lambda i: (0, 0)),
            pl.BlockSpec((1, HIDDEN), lambda i: (0, 0)),
            pl.BlockSpec((HIDDEN, IN_C * OUT_C), lambda i: (0, 0)),
            pl.BlockSpec((1, IN_C * OUT_C), lambda i: (0, 0)),
            pl.BlockSpec((IN_C, IN_C * OUT_C), lambda i: (0, 0)),
            pl.BlockSpec((4 * OUT_C, OUT_C), lambda i: (0, 0)),
        ],
        out_specs=pl.BlockSpec((BE, AUG), lambda i: (i, 0)),
        out_shape=jax.ShapeDtypeStruct((ECH, AUG), jnp.float32),
    )(ea_pad, xj, W1, b1.reshape(1, HIDDEN), W2,
      b2.reshape(1, IN_C * OUT_C), R, S)


BN = 2000


def _node_body(p_ref, q_ref, x_ref, root_ref, bias_ref, g_ref, beta_ref,
               Wres_ref, bres_ref, o_ref):
    p = (p_ref[0] + p_ref[1]) + (q_ref[0] + q_ref[1])
    summed = p[:, 0:OUT_C]
    cnt = p[:, OUT_C:OUT_C + 1]
    aggr = summed / jnp.maximum(cnt, 1.0)
    xb = x_ref[...]
    out = aggr + jnp.dot(xb, root_ref[...],
                         preferred_element_type=jnp.float32) + bias_ref[...]
    mu = jnp.mean(out, axis=1, keepdims=True)
    var = jnp.mean((out - mu) * (out - mu), axis=1, keepdims=True)
    out = (out - mu) * lax.rsqrt(var + 1e-5) * g_ref[...] + beta_ref[...]
    res = jnp.dot(xb, Wres_ref[...],
                  preferred_element_type=jnp.float32) + bres_ref[...]
    t = out + res
    o_ref[...] = t * jax.nn.sigmoid(t)


def _tc_node(pa, pb, x, root, bias, ln_gamma, ln_beta, Wres, bres):
    grid = (N // BN,)
    return pl.pallas_call(
        _node_body,
        grid=grid,
        in_specs=[
            pl.BlockSpec((2, BN, AUG), lambda i: (0, i, 0)),
            pl.BlockSpec((2, BN, AUG), lambda i: (0, i, 0)),
            pl.BlockSpec((BN, IN_C), lambda i: (i, 0)),
            pl.BlockSpec((IN_C, OUT_C), lambda i: (0, 0)),
            pl.BlockSpec((1, OUT_C), lambda i: (0, 0)),
            pl.BlockSpec((1, OUT_C), lambda i: (0, 0)),
            pl.BlockSpec((1, OUT_C), lambda i: (0, 0)),
            pl.BlockSpec((IN_C, OUT_C), lambda i: (0, 0)),
            pl.BlockSpec((1, OUT_C), lambda i: (0, 0)),
        ],
        out_specs=pl.BlockSpec((BN, OUT_C), lambda i: (i, 0)),
        out_shape=jax.ShapeDtypeStruct((N, OUT_C), jnp.float32),
    )(pa, pb, x, root, bias.reshape(1, OUT_C), ln_gamma.reshape(1, OUT_C),
      ln_beta.reshape(1, OUT_C), Wres, bres.reshape(1, OUT_C))


def kernel(x, edge_attr, W1, b1, W2, b2, root, bias, ln_gamma, ln_beta,
           Wres, bres, edge_index):
    pad = E_PAD - E
    src_pad = jnp.concatenate(
        [edge_index[0], jnp.zeros((pad,), jnp.int32)])
    dst_pad = jnp.concatenate(
        [edge_index[1], jnp.full((pad,), N, jnp.int32)])
    ea_pad = jnp.concatenate(
        [edge_attr, jnp.zeros((pad, EDGE_DIM), jnp.float32)])
    dst2 = dst_pad.reshape(E_PAD // CH, CH)
    src2 = src_pad.reshape(E_PAD // CH, CH)
    zrows = jnp.zeros((N_ACC, AUG), jnp.float32)

    msgs = []
    for ci in range(NPIPE):
        xj_c = _sc_gather_chunk(x, src2, ci)
        msgs.append(_tc_edge(ea_pad, xj_c, W1, b1, W2, b2, ci))
    pa = _sc_scatter(msgs[0:2], dst2, zrows, (0, 1))
    pb = _sc_scatter(msgs[2:4], dst2, zrows, (2, 3))
    return _tc_node(pa, pb, x, root, bias, ln_gamma, ln_beta, Wres, bres)
